# baseline (speedup 1.0000x reference)
import functools

import numpy as np
import jax
import jax.numpy as jnp
from jax import lax
from jax.experimental import pallas as pl
from jax.experimental.pallas import tpu as pltpu
from jax.experimental.pallas import tpu_sc as plsc

_PC_RANGE = np.array([0.0, -3.14159265, -0.4363, 50.0, 3.14159265, 0.0524],
                     dtype=np.float32)
_GRID = (240, 180, 32)
_NCLASS = 20
_N = 480000
_NVOX = 240 * 180 * 32

_NPAD = 480256
_ROWS = 3752
_RB = 536
_NBLK = _ROWS // _RB

_NSC = 2
_NTILE = 16
_HALF_VOX = _NVOX // 2
_QUART = _HALF_VOX // 2
_HIST_W = _QUART * 5
_TRASH = 2048
_SPW = _HIST_W + _TRASH
_KPT = _N // _NTILE
_SSTRIP = 10000
_NSTRIP = _KPT // _SSTRIP
_ZSL = _SPW // _NTILE
_RSL = _HIST_W // _NTILE


def _points_body(xyz_ref, inten_ref, lab_ref, fea_ref, grid_ref, pack_ref):
    x = xyz_ref[0]
    y = xyz_ref[1]
    z = xyz_ref[2]
    x2y2 = x * x + y * y
    rho = jnp.sqrt(x2y2 + z * z + 1e-12)
    phi = jnp.arctan2(y, x)
    pitch = jnp.arctan2(z, jnp.sqrt(x2y2) + 1e-12)

    gis = []
    cens = []
    sphs = (rho, phi, pitch)
    for a in range(3):
        mn = _PC_RANGE[a]
        mx = _PC_RANGE[3 + a]
        iv = np.float32((mx - mn) / np.float32(_GRID[a]))
        hi = np.float32(mx - np.float32(1e-4) * iv)
        v = sphs[a]
        c = jnp.clip(v, mn, hi)
        gi = jnp.floor((c - mn) / iv).astype(jnp.int32)
        gi = jnp.clip(gi, 0, _GRID[a] - 1)
        center = (gi.astype(jnp.float32) + 0.5) * iv + mn
        gis.append(gi)
        cens.append(v - center)

    fea_ref[0] = cens[0]
    fea_ref[1] = cens[1]
    fea_ref[2] = cens[2]
    fea_ref[3] = rho
    fea_ref[4] = phi
    fea_ref[5] = pitch
    fea_ref[6] = x
    fea_ref[7] = y
    fea_ref[8] = inten_ref[...]
    grid_ref[0] = gis[0]
    grid_ref[1] = gis[1]
    grid_ref[2] = gis[2]

    flat = (gis[0] * _GRID[1] + gis[1]) * _GRID[2] + gis[2]
    lab = lab_ref[...]
    in0 = flat < _HALF_VOX
    rel = jnp.where(in0, flat, flat - _HALF_VOX)
    low = rel < _QUART
    g = jnp.where(low, rel, rel - _QUART)
    bitpos = jnp.where(low, lab * 4, 80 + lab * 4)
    word = (bitpos >> 5) * _QUART + g
    half_bit = jnp.where(in0, 0, 1 << 26)
    packed = half_bit | (word << 3) | ((bitpos & 31) >> 2)
    pack_ref[...] = packed.reshape(_RB * 128)


def _points_call(xyz_t, inten2, lab2):
    return pl.pallas_call(
        _points_body,
        grid=(_NBLK,),
        in_specs=[
            pl.BlockSpec((3, _RB, 128), lambda i: (0, i, 0)),
            pl.BlockSpec((_RB, 128), lambda i: (i, 0)),
            pl.BlockSpec((_RB, 128), lambda i: (i, 0)),
        ],
        out_specs=[
            pl.BlockSpec((9, _RB, 128), lambda i: (0, i, 0)),
            pl.BlockSpec((3, _RB, 128), lambda i: (0, i, 0)),
            pl.BlockSpec((_RB * 128,), lambda i: (i,)),
        ],
        out_shape=[
            jax.ShapeDtypeStruct((9, _ROWS, 128), jnp.float32),
            jax.ShapeDtypeStruct((3, _ROWS, 128), jnp.int32),
            jax.ShapeDtypeStruct((_NPAD,), jnp.int32),
        ],
    )(xyz_t, inten2, lab2)


def _sc_hist_body(pack_hbm, zeros_hbm, out_hbm, hist_sh, b_v, v_v):
    cid = lax.axis_index("c")
    sid = lax.axis_index("s")
    pltpu.sync_copy(zeros_hbm, v_v)
    nfull = _ZSL // _SSTRIP
    for j in range(nfull):
        pltpu.sync_copy(v_v,
                        hist_sh.at[pl.ds(sid * _ZSL + j * _SSTRIP, _SSTRIP)])
    rem = _ZSL - nfull * _SSTRIP
    if rem:
        pltpu.sync_copy(v_v.at[pl.ds(0, rem)],
                        hist_sh.at[pl.ds(sid * _ZSL + nfull * _SSTRIP, rem)])
    plsc.subcore_barrier()

    half_flag = cid << 26
    for s in range(_NSTRIP):
        pltpu.sync_copy(
            pack_hbm.at[pl.ds(sid * _KPT + s * _SSTRIP, _SSTRIP)], b_v)

        def body(i, carry):
            p = b_v[pl.ds(i * 16, 16)]
            mine = (p & (1 << 26)) == half_flag
            word = (p >> 3) & ((1 << 23) - 1)
            tr = _HIST_W + (p & (_TRASH - 1))
            b_v[pl.ds(i * 16, 16)] = jnp.where(mine, word, tr)
            v_v[pl.ds(i * 16, 16)] = jnp.left_shift(
                jnp.ones_like(p), (p & 7) * 4)
            return carry

        lax.fori_loop(0, _SSTRIP // 16, body, 0)
        pltpu.sync_copy(v_v, hist_sh.at[b_v], add=True)
    plsc.subcore_barrier()

    nrf = _RSL // _SSTRIP
    for j in range(nrf):
        pltpu.sync_copy(hist_sh.at[pl.ds(sid * _RSL + j * _SSTRIP, _SSTRIP)],
                        b_v)
        pltpu.sync_copy(
            b_v,
            out_hbm.at[pl.ds(cid * _HIST_W + sid * _RSL + j * _SSTRIP,
                             _SSTRIP)])
    rrem = _RSL - nrf * _SSTRIP
    if rrem:
        pltpu.sync_copy(
            hist_sh.at[pl.ds(sid * _RSL + nrf * _SSTRIP, rrem)],
            b_v.at[pl.ds(0, rrem)])
        pltpu.sync_copy(
            b_v.at[pl.ds(0, rrem)],
            out_hbm.at[pl.ds(cid * _HIST_W + sid * _RSL + nrf * _SSTRIP,
                             rrem)])


def _sc_hist_call(pack, zeros):
    mesh = plsc.VectorSubcoreMesh(core_axis_name="c", subcore_axis_name="s",
                                  num_cores=_NSC, num_subcores=_NTILE)
    return pl.kernel(
        _sc_hist_body,
        out_type=jax.ShapeDtypeStruct((_NSC * _HIST_W,), jnp.int32),
        mesh=mesh,
        scratch_types=[
            pltpu.VMEM_SHARED((_SPW,), jnp.int32),
            pltpu.VMEM((_SSTRIP,), jnp.int32),
            pltpu.VMEM((_SSTRIP,), jnp.int32),
        ],
    )(pack, zeros)


def _vote_body(cnt_ref, out_ref):
    w = [cnt_ref[0, p] for p in range(5)]

    for parity in (0, 1):
        def cnt(c):
            bp = parity * 80 + 4 * c
            return (w[bp >> 5] >> (bp & 31)) & 15

        best = cnt(1)
        bestc = jnp.full_like(best, 1)
        for c in range(2, _NCLASS):
            cc = cnt(c)
            gt = cc > best
            best = jnp.where(gt, cc, best)
            bestc = jnp.where(gt, c, bestc)
        nonempty = (best > 0) | (cnt(0) > 0)
        out_ref[0, parity] = jnp.where(nonempty, bestc, 0)


def _vote_call(counts4):
    return pl.pallas_call(
        _vote_body,
        grid=(_NSC, 4),
        in_specs=[pl.BlockSpec((1, 5, 675, 128), lambda c, j: (c, 0, 0, j))],
        out_specs=pl.BlockSpec((1, 2, 675, 128), lambda c, j: (c, 0, 0, j)),
        out_shape=jax.ShapeDtypeStruct((_NSC, 2, 675, 512), jnp.int32),
    )(counts4)


def kernel(xyz, intensity, labels):
    pad = _NPAD - _N
    xyz_t = jnp.pad(xyz, ((0, pad), (0, 0))).T.reshape(3, _ROWS, 128)
    inten2 = jnp.pad(intensity, (0, pad)).reshape(_ROWS, 128)
    lab2 = jnp.pad(labels, (0, pad)).reshape(_ROWS, 128)

    fea, grid, pack = _points_call(xyz_t, inten2, lab2)

    zeros = jnp.zeros((_SSTRIP,), jnp.int32)
    counts = _sc_hist_call(pack, zeros)
    counts4 = counts.reshape(_NSC, 5, 675, 512)
    voted = _vote_call(counts4)
    processed_label = voted.reshape(_GRID)

    return_fea = fea.reshape(9, _NPAD)[:, :_N].T
    grid_ind = grid.reshape(3, _NPAD)[:, :_N].T
    return return_fea, processed_label, grid_ind


# speedup vs baseline: 1.0182x; 1.0182x over previous
import functools

import numpy as np
import jax
import jax.numpy as jnp
from jax import lax
from jax.experimental import pallas as pl
from jax.experimental.pallas import tpu as pltpu
from jax.experimental.pallas import tpu_sc as plsc

_PC_RANGE = np.array([0.0, -3.14159265, -0.4363, 50.0, 3.14159265, 0.0524],
                     dtype=np.float32)
_GRID = (240, 180, 32)
_NCLASS = 20
_N = 480000
_NVOX = 240 * 180 * 32

_NPAD = 480256
_ROWS = 3752
_RB = 536
_NBLK = _ROWS // _RB

_NSC = 2
_NTILE = 16
_HALF_VOX = _NVOX // 2
_QUART = _HALF_VOX // 2
_HIST_W = _QUART * 5
_TRASH = 2048
_SPW = _HIST_W + _TRASH
_KPT = _N // _NTILE
_SSTRIP = 10000
_NSTRIP = _KPT // _SSTRIP
_ZSL = _SPW // _NTILE
_RSL = _HIST_W // _NTILE


def _points_body(xyz_ref, inten_ref, lab_ref, fea_ref, grid_ref, bins_ref,
                 vals_ref):
    x = xyz_ref[0]
    y = xyz_ref[1]
    z = xyz_ref[2]
    x2y2 = x * x + y * y
    rho = jnp.sqrt(x2y2 + z * z + 1e-12)
    phi = jnp.arctan2(y, x)
    pitch = jnp.arctan2(z, jnp.sqrt(x2y2) + 1e-12)

    gis = []
    cens = []
    sphs = (rho, phi, pitch)
    for a in range(3):
        mn = _PC_RANGE[a]
        mx = _PC_RANGE[3 + a]
        iv = np.float32((mx - mn) / np.float32(_GRID[a]))
        hi = np.float32(mx - np.float32(1e-4) * iv)
        v = sphs[a]
        c = jnp.clip(v, mn, hi)
        gi = jnp.floor((c - mn) / iv).astype(jnp.int32)
        gi = jnp.clip(gi, 0, _GRID[a] - 1)
        center = (gi.astype(jnp.float32) + 0.5) * iv + mn
        gis.append(gi)
        cens.append(v - center)

    fea_ref[0] = cens[0]
    fea_ref[1] = cens[1]
    fea_ref[2] = cens[2]
    fea_ref[3] = rho
    fea_ref[4] = phi
    fea_ref[5] = pitch
    fea_ref[6] = x
    fea_ref[7] = y
    fea_ref[8] = inten_ref[...]
    grid_ref[0] = gis[0]
    grid_ref[1] = gis[1]
    grid_ref[2] = gis[2]

    flat = (gis[0] * _GRID[1] + gis[1]) * _GRID[2] + gis[2]
    lab = lab_ref[...]
    in0 = flat < _HALF_VOX
    rel = jnp.where(in0, flat, flat - _HALF_VOX)
    low = rel < _QUART
    g = jnp.where(low, rel, rel - _QUART)
    bitpos = jnp.where(low, lab * 4, 80 + lab * 4)
    word = (bitpos >> 5) * _QUART + g
    trash = _HIST_W + (flat & (_TRASH - 1))
    bins_ref[0] = jnp.where(in0, word, trash)
    bins_ref[1] = jnp.where(in0, trash, word)
    vals_ref[...] = jnp.left_shift(jnp.ones_like(flat), bitpos & 31)


def _points_call(xyz_t, inten2, lab2):
    return pl.pallas_call(
        _points_body,
        grid=(_NBLK,),
        in_specs=[
            pl.BlockSpec((3, _RB, 128), lambda i: (0, i, 0)),
            pl.BlockSpec((_RB, 128), lambda i: (i, 0)),
            pl.BlockSpec((_RB, 128), lambda i: (i, 0)),
        ],
        out_specs=[
            pl.BlockSpec((9, _RB, 128), lambda i: (0, i, 0)),
            pl.BlockSpec((3, _RB, 128), lambda i: (0, i, 0)),
            pl.BlockSpec((2, _RB, 128), lambda i: (0, i, 0)),
            pl.BlockSpec((_RB, 128), lambda i: (i, 0)),
        ],
        out_shape=[
            jax.ShapeDtypeStruct((9, _ROWS, 128), jnp.float32),
            jax.ShapeDtypeStruct((3, _ROWS, 128), jnp.int32),
            jax.ShapeDtypeStruct((2, _ROWS, 128), jnp.int32),
            jax.ShapeDtypeStruct((_ROWS, 128), jnp.int32),
        ],
    )(xyz_t, inten2, lab2)


def _sc_hist_body(bins_hbm, vals_hbm, zeros_hbm, out_hbm, hist_sh, b_v, v_v):
    cid = lax.axis_index("c")
    sid = lax.axis_index("s")
    pltpu.sync_copy(zeros_hbm, v_v)
    nfull = _ZSL // _SSTRIP
    for j in range(nfull):
        pltpu.sync_copy(v_v,
                        hist_sh.at[pl.ds(sid * _ZSL + j * _SSTRIP, _SSTRIP)])
    rem = _ZSL - nfull * _SSTRIP
    if rem:
        pltpu.sync_copy(v_v.at[pl.ds(0, rem)],
                        hist_sh.at[pl.ds(sid * _ZSL + nfull * _SSTRIP, rem)])
    plsc.subcore_barrier()

    for s in range(_NSTRIP):
        pltpu.sync_copy(
            bins_hbm.at[pl.ds(cid * _NPAD + sid * _KPT + s * _SSTRIP,
                              _SSTRIP)], b_v)
        pltpu.sync_copy(
            vals_hbm.at[pl.ds(sid * _KPT + s * _SSTRIP, _SSTRIP)], v_v)
        pltpu.sync_copy(v_v, hist_sh.at[b_v], add=True)
    plsc.subcore_barrier()

    nrf = _RSL // _SSTRIP
    for j in range(nrf):
        pltpu.sync_copy(hist_sh.at[pl.ds(sid * _RSL + j * _SSTRIP, _SSTRIP)],
                        b_v)
        pltpu.sync_copy(
            b_v,
            out_hbm.at[pl.ds(cid * _HIST_W + sid * _RSL + j * _SSTRIP,
                             _SSTRIP)])
    rrem = _RSL - nrf * _SSTRIP
    if rrem:
        pltpu.sync_copy(
            hist_sh.at[pl.ds(sid * _RSL + nrf * _SSTRIP, rrem)],
            b_v.at[pl.ds(0, rrem)])
        pltpu.sync_copy(
            b_v.at[pl.ds(0, rrem)],
            out_hbm.at[pl.ds(cid * _HIST_W + sid * _RSL + nrf * _SSTRIP,
                             rrem)])


def _sc_hist_call(bins, vals, zeros):
    mesh = plsc.VectorSubcoreMesh(core_axis_name="c", subcore_axis_name="s",
                                  num_cores=_NSC, num_subcores=_NTILE)
    return pl.kernel(
        _sc_hist_body,
        out_type=jax.ShapeDtypeStruct((_NSC * _HIST_W,), jnp.int32),
        mesh=mesh,
        scratch_types=[
            pltpu.VMEM_SHARED((_SPW,), jnp.int32),
            pltpu.VMEM((_SSTRIP,), jnp.int32),
            pltpu.VMEM((_SSTRIP,), jnp.int32),
        ],
    )(bins, vals, zeros)


def _vote_body(cnt_ref, out_ref):
    w = [cnt_ref[0, p] for p in range(5)]

    for parity in (0, 1):
        def cnt(c):
            bp = parity * 80 + 4 * c
            return (w[bp >> 5] >> (bp & 31)) & 15

        best = cnt(1)
        bestc = jnp.full_like(best, 1)
        for c in range(2, _NCLASS):
            cc = cnt(c)
            gt = cc > best
            best = jnp.where(gt, cc, best)
            bestc = jnp.where(gt, c, bestc)
        nonempty = (best > 0) | (cnt(0) > 0)
        out_ref[0, parity] = jnp.where(nonempty, bestc, 0)


def _vote_call(counts4):
    return pl.pallas_call(
        _vote_body,
        grid=(_NSC, 4),
        in_specs=[pl.BlockSpec((1, 5, 675, 128), lambda c, j: (c, 0, 0, j))],
        out_specs=pl.BlockSpec((1, 2, 675, 128), lambda c, j: (c, 0, 0, j)),
        out_shape=jax.ShapeDtypeStruct((_NSC, 2, 675, 512), jnp.int32),
    )(counts4)


def kernel(xyz, intensity, labels):
    pad = _NPAD - _N
    xyz_t = jnp.pad(xyz, ((0, pad), (0, 0))).T.reshape(3, _ROWS, 128)
    inten2 = jnp.pad(intensity, (0, pad)).reshape(_ROWS, 128)
    lab2 = jnp.pad(labels, (0, pad)).reshape(_ROWS, 128)

    fea, grid, bins, vals = _points_call(xyz_t, inten2, lab2)

    zeros = jnp.zeros((_SSTRIP,), jnp.int32)
    counts = _sc_hist_call(bins.reshape(_NSC * _NPAD), vals.reshape(_NPAD),
                           zeros)
    fea, grid, counts = lax.optimization_barrier((fea, grid, counts))
    counts4 = counts.reshape(_NSC, 5, 675, 512)
    voted = _vote_call(counts4)
    processed_label = voted.reshape(_GRID)

    return_fea = fea.reshape(9, _NPAD)[:, :_N].T
    grid_ind = grid.reshape(3, _NPAD)[:, :_N].T
    return return_fea, processed_label, grid_ind


# speedup vs baseline: 1.2211x; 1.1993x over previous
import functools

import numpy as np
import jax
import jax.numpy as jnp
from jax import lax
from jax.experimental import pallas as pl
from jax.experimental.pallas import tpu as pltpu
from jax.experimental.pallas import tpu_sc as plsc

_PC_RANGE = np.array([0.0, -3.14159265, -0.4363, 50.0, 3.14159265, 0.0524],
                     dtype=np.float32)
_GRID = (240, 180, 32)
_NCLASS = 20
_N = 480000
_NVOX = 240 * 180 * 32

_ROWS = 3750
_RB = 752
_NBLK = -(-_ROWS // _RB)

_NSC = 2
_NTILE = 16
_HALF_VOX = _NVOX // 2
_QUART = _HALF_VOX // 2
_HIST_W = _QUART * 5
_TRASH = 2048
_SPW = _HIST_W + _TRASH
_KPT = _N // _NTILE
_SSTRIP = 10000
_NSTRIP = _KPT // _SSTRIP
_ZSL = _SPW // _NTILE
_RSL = _HIST_W // _NTILE


def _points_body(xyz_ref, inten_ref, lab_ref, fea_ref, grid_ref, bins_ref,
                 vals_ref):
    x = xyz_ref[0]
    y = xyz_ref[1]
    z = xyz_ref[2]
    x2y2 = x * x + y * y
    rho = jnp.sqrt(x2y2 + z * z + 1e-12)
    phi = jnp.arctan2(y, x)
    pitch = jnp.arctan2(z, jnp.sqrt(x2y2) + 1e-12)

    gis = []
    cens = []
    sphs = (rho, phi, pitch)
    for a in range(3):
        mn = _PC_RANGE[a]
        mx = _PC_RANGE[3 + a]
        iv = np.float32((mx - mn) / np.float32(_GRID[a]))
        hi = np.float32(mx - np.float32(1e-4) * iv)
        v = sphs[a]
        c = jnp.clip(v, mn, hi)
        gi = jnp.floor((c - mn) / iv).astype(jnp.int32)
        gi = jnp.clip(gi, 0, _GRID[a] - 1)
        center = (gi.astype(jnp.float32) + 0.5) * iv + mn
        gis.append(gi)
        cens.append(v - center)

    fea_ref[0] = cens[0]
    fea_ref[1] = cens[1]
    fea_ref[2] = cens[2]
    fea_ref[3] = rho
    fea_ref[4] = phi
    fea_ref[5] = pitch
    fea_ref[6] = x
    fea_ref[7] = y
    fea_ref[8] = inten_ref[...]
    grid_ref[0] = gis[0]
    grid_ref[1] = gis[1]
    grid_ref[2] = gis[2]

    flat = (gis[0] * _GRID[1] + gis[1]) * _GRID[2] + gis[2]
    lab = lab_ref[...]
    in0 = flat < _HALF_VOX
    rel = jnp.where(in0, flat, flat - _HALF_VOX)
    low = rel < _QUART
    g = jnp.where(low, rel, rel - _QUART)
    bitpos = jnp.where(low, lab * 4, 80 + lab * 4)
    word = (bitpos >> 5) * _QUART + g
    trash = _HIST_W + (flat & (_TRASH - 1))
    bins_ref[0] = jnp.where(in0, word, trash)
    bins_ref[1] = jnp.where(in0, trash, word)
    vals_ref[...] = jnp.left_shift(jnp.ones_like(flat), bitpos & 31)


def _points_call(xyz_t, inten2, lab2):
    return pl.pallas_call(
        _points_body,
        grid=(_NBLK,),
        in_specs=[
            pl.BlockSpec((3, _RB, 128), lambda i: (0, i, 0)),
            pl.BlockSpec((_RB, 128), lambda i: (i, 0)),
            pl.BlockSpec((_RB, 128), lambda i: (i, 0)),
        ],
        out_specs=[
            pl.BlockSpec((9, _RB, 128), lambda i: (0, i, 0)),
            pl.BlockSpec((3, _RB, 128), lambda i: (0, i, 0)),
            pl.BlockSpec((2, _RB, 128), lambda i: (0, i, 0)),
            pl.BlockSpec((_RB, 128), lambda i: (i, 0)),
        ],
        out_shape=[
            jax.ShapeDtypeStruct((9, _ROWS, 128), jnp.float32),
            jax.ShapeDtypeStruct((3, _ROWS, 128), jnp.int32),
            jax.ShapeDtypeStruct((2, _ROWS, 128), jnp.int32),
            jax.ShapeDtypeStruct((_ROWS, 128), jnp.int32),
        ],
    )(xyz_t, inten2, lab2)


def _sc_hist_body(bins_hbm, vals_hbm, zeros_hbm, out_hbm, hist_sh, b_v, v_v):
    cid = lax.axis_index("c")
    sid = lax.axis_index("s")
    pltpu.sync_copy(zeros_hbm, v_v)
    nfull = _ZSL // _SSTRIP
    for j in range(nfull):
        pltpu.sync_copy(v_v,
                        hist_sh.at[pl.ds(sid * _ZSL + j * _SSTRIP, _SSTRIP)])
    rem = _ZSL - nfull * _SSTRIP
    if rem:
        pltpu.sync_copy(v_v.at[pl.ds(0, rem)],
                        hist_sh.at[pl.ds(sid * _ZSL + nfull * _SSTRIP, rem)])
    plsc.subcore_barrier()

    for s in range(_NSTRIP):
        pltpu.sync_copy(
            bins_hbm.at[pl.ds(cid * _N + sid * _KPT + s * _SSTRIP,
                              _SSTRIP)], b_v)
        pltpu.sync_copy(
            vals_hbm.at[pl.ds(sid * _KPT + s * _SSTRIP, _SSTRIP)], v_v)
        pltpu.sync_copy(v_v, hist_sh.at[b_v], add=True)
    plsc.subcore_barrier()

    nrf = _RSL // _SSTRIP
    for j in range(nrf):
        pltpu.sync_copy(hist_sh.at[pl.ds(sid * _RSL + j * _SSTRIP, _SSTRIP)],
                        b_v)
        pltpu.sync_copy(
            b_v,
            out_hbm.at[pl.ds(cid * _HIST_W + sid * _RSL + j * _SSTRIP,
                             _SSTRIP)])
    rrem = _RSL - nrf * _SSTRIP
    if rrem:
        pltpu.sync_copy(
            hist_sh.at[pl.ds(sid * _RSL + nrf * _SSTRIP, rrem)],
            b_v.at[pl.ds(0, rrem)])
        pltpu.sync_copy(
            b_v.at[pl.ds(0, rrem)],
            out_hbm.at[pl.ds(cid * _HIST_W + sid * _RSL + nrf * _SSTRIP,
                             rrem)])


def _sc_hist_call(bins, vals, zeros):
    mesh = plsc.VectorSubcoreMesh(core_axis_name="c", subcore_axis_name="s",
                                  num_cores=_NSC, num_subcores=_NTILE)
    return pl.kernel(
        _sc_hist_body,
        out_type=jax.ShapeDtypeStruct((_NSC * _HIST_W,), jnp.int32),
        mesh=mesh,
        scratch_types=[
            pltpu.VMEM_SHARED((_SPW,), jnp.int32),
            pltpu.VMEM((_SSTRIP,), jnp.int32),
            pltpu.VMEM((_SSTRIP,), jnp.int32),
        ],
    )(bins, vals, zeros)


def _vote_body(cnt_ref, out_ref):
    w = [cnt_ref[0, p] for p in range(5)]

    for parity in (0, 1):
        def cnt(c):
            bp = parity * 80 + 4 * c
            return (w[bp >> 5] >> (bp & 31)) & 15

        best = cnt(1)
        bestc = jnp.full_like(best, 1)
        for c in range(2, _NCLASS):
            cc = cnt(c)
            gt = cc > best
            best = jnp.where(gt, cc, best)
            bestc = jnp.where(gt, c, bestc)
        nonempty = (best > 0) | (cnt(0) > 0)
        out_ref[0, parity] = jnp.where(nonempty, bestc, 0)


def _vote_call(counts4):
    return pl.pallas_call(
        _vote_body,
        grid=(_NSC, 4),
        in_specs=[pl.BlockSpec((1, 5, 675, 128), lambda c, j: (c, 0, 0, j))],
        out_specs=pl.BlockSpec((1, 2, 675, 128), lambda c, j: (c, 0, 0, j)),
        out_shape=jax.ShapeDtypeStruct((_NSC, 2, 675, 512), jnp.int32),
    )(counts4)


def kernel(xyz, intensity, labels):
    xyz_t = xyz.T.reshape(3, _ROWS, 128)
    inten2 = intensity.reshape(_ROWS, 128)
    lab2 = labels.reshape(_ROWS, 128)

    fea, grid, bins, vals = _points_call(xyz_t, inten2, lab2)

    zeros = jnp.zeros((_SSTRIP,), jnp.int32)
    counts = _sc_hist_call(bins.reshape(_NSC * _N), vals.reshape(_N),
                           zeros)
    fea, grid, counts = lax.optimization_barrier((fea, grid, counts))
    counts4 = counts.reshape(_NSC, 5, 675, 512)
    voted = _vote_call(counts4)
    processed_label = voted.reshape(_GRID)

    return_fea = fea.reshape(9, _N).T
    grid_ind = grid.reshape(3, _N).T
    return return_fea, processed_label, grid_ind


# speedup vs baseline: 1.3346x; 1.0929x over previous
import functools

import numpy as np
import jax
import jax.numpy as jnp
from jax import lax
from jax.experimental import pallas as pl
from jax.experimental.pallas import tpu as pltpu
from jax.experimental.pallas import tpu_sc as plsc

_PC_RANGE = np.array([0.0, -3.14159265, -0.4363, 50.0, 3.14159265, 0.0524],
                     dtype=np.float32)
_GRID = (240, 180, 32)
_NCLASS = 20
_N = 480000
_NVOX = 240 * 180 * 32

_ROWS = 3750
_RB = 752
_NBLK = -(-_ROWS // _RB)

_NSC = 2
_NTILE = 16
_HALF_VOX = _NVOX // 2
_QUART = _HALF_VOX // 2
_HIST_W = _QUART * 5
_TRASH = 2048
_SPW = _HIST_W + _TRASH
_KPT = _N // _NTILE
_SSTRIP = 10000
_NSTRIP = _KPT // _SSTRIP
_ZSL = _SPW // _NTILE
_RSL = _HIST_W // _NTILE


def _points_body(xyz_ref, inten_ref, lab_ref, fea_ref, grid_ref, bins_ref,
                 vals_ref):
    x = xyz_ref[0]
    y = xyz_ref[1]
    z = xyz_ref[2]
    x2y2 = x * x + y * y
    rho = jnp.sqrt(x2y2 + z * z + 1e-12)
    phi = jnp.arctan2(y, x)
    pitch = jnp.arctan2(z, jnp.sqrt(x2y2) + 1e-12)

    gis = []
    cens = []
    sphs = (rho, phi, pitch)
    for a in range(3):
        mn = _PC_RANGE[a]
        mx = _PC_RANGE[3 + a]
        iv = np.float32((mx - mn) / np.float32(_GRID[a]))
        hi = np.float32(mx - np.float32(1e-4) * iv)
        v = sphs[a]
        c = jnp.clip(v, mn, hi)
        gi = jnp.floor((c - mn) / iv).astype(jnp.int32)
        gi = jnp.clip(gi, 0, _GRID[a] - 1)
        center = (gi.astype(jnp.float32) + 0.5) * iv + mn
        gis.append(gi)
        cens.append(v - center)

    fea_ref[0] = cens[0]
    fea_ref[1] = cens[1]
    fea_ref[2] = cens[2]
    fea_ref[3] = rho
    fea_ref[4] = phi
    fea_ref[5] = pitch
    fea_ref[6] = x
    fea_ref[7] = y
    fea_ref[8] = inten_ref[...]
    grid_ref[0] = gis[0]
    grid_ref[1] = gis[1]
    grid_ref[2] = gis[2]

    flat = (gis[0] * _GRID[1] + gis[1]) * _GRID[2] + gis[2]
    lab = lab_ref[...]
    in0 = flat < _HALF_VOX
    rel = jnp.where(in0, flat, flat - _HALF_VOX)
    low = rel < _QUART
    g = jnp.where(low, rel, rel - _QUART)
    bitpos = jnp.where(low, lab * 4, 80 + lab * 4)
    word = (bitpos >> 5) * _QUART + g
    trash = _HIST_W + (flat & (_TRASH - 1))
    bins_ref[0] = jnp.where(in0, word, trash)
    bins_ref[1] = jnp.where(in0, trash, word)
    vals_ref[...] = jnp.left_shift(jnp.ones_like(flat), bitpos & 31)


def _points_call(xyz_t, inten2, lab2):
    return pl.pallas_call(
        _points_body,
        grid=(_NBLK,),
        in_specs=[
            pl.BlockSpec((3, _RB, 128), lambda i: (0, i, 0)),
            pl.BlockSpec((_RB, 128), lambda i: (i, 0)),
            pl.BlockSpec((_RB, 128), lambda i: (i, 0)),
        ],
        out_specs=[
            pl.BlockSpec((9, _RB, 128), lambda i: (0, i, 0)),
            pl.BlockSpec((3, _RB, 128), lambda i: (0, i, 0)),
            pl.BlockSpec((2, _RB, 128), lambda i: (0, i, 0)),
            pl.BlockSpec((_RB, 128), lambda i: (i, 0)),
        ],
        out_shape=[
            jax.ShapeDtypeStruct((9, _ROWS, 128), jnp.float32),
            jax.ShapeDtypeStruct((3, _ROWS, 128), jnp.int32),
            jax.ShapeDtypeStruct((2, _ROWS, 128), jnp.int32),
            jax.ShapeDtypeStruct((_ROWS, 128), jnp.int32),
        ],
    )(xyz_t, inten2, lab2)


def _sc_hist_body(bins_hbm, vals_hbm, zeros_hbm, out_hbm, hist_sh, b_v, v_v):
    cid = lax.axis_index("c")
    sid = lax.axis_index("s")
    pltpu.sync_copy(zeros_hbm, v_v)
    nfull = _ZSL // _SSTRIP
    for j in range(nfull):
        pltpu.sync_copy(v_v,
                        hist_sh.at[pl.ds(sid * _ZSL + j * _SSTRIP, _SSTRIP)])
    rem = _ZSL - nfull * _SSTRIP
    if rem:
        pltpu.sync_copy(v_v.at[pl.ds(0, rem)],
                        hist_sh.at[pl.ds(sid * _ZSL + nfull * _SSTRIP, rem)])
    plsc.subcore_barrier()

    for s in range(_NSTRIP):
        pltpu.sync_copy(
            bins_hbm.at[pl.ds(cid * _N + sid * _KPT + s * _SSTRIP,
                              _SSTRIP)], b_v)
        pltpu.sync_copy(
            vals_hbm.at[pl.ds(sid * _KPT + s * _SSTRIP, _SSTRIP)], v_v)
        pltpu.sync_copy(v_v, hist_sh.at[b_v], add=True)
    plsc.subcore_barrier()

    nrf = _RSL // _SSTRIP
    for j in range(nrf):
        pltpu.sync_copy(hist_sh.at[pl.ds(sid * _RSL + j * _SSTRIP, _SSTRIP)],
                        b_v)
        pltpu.sync_copy(
            b_v,
            out_hbm.at[pl.ds(cid * _HIST_W + sid * _RSL + j * _SSTRIP,
                             _SSTRIP)])
    rrem = _RSL - nrf * _SSTRIP
    if rrem:
        pltpu.sync_copy(
            hist_sh.at[pl.ds(sid * _RSL + nrf * _SSTRIP, rrem)],
            b_v.at[pl.ds(0, rrem)])
        pltpu.sync_copy(
            b_v.at[pl.ds(0, rrem)],
            out_hbm.at[pl.ds(cid * _HIST_W + sid * _RSL + nrf * _SSTRIP,
                             rrem)])


def _sc_hist_call(bins, vals, zeros):
    mesh = plsc.VectorSubcoreMesh(core_axis_name="c", subcore_axis_name="s",
                                  num_cores=_NSC, num_subcores=_NTILE)
    return pl.kernel(
        _sc_hist_body,
        out_type=jax.ShapeDtypeStruct((_NSC * _HIST_W,), jnp.int32),
        mesh=mesh,
        scratch_types=[
            pltpu.VMEM_SHARED((_SPW,), jnp.int32),
            pltpu.VMEM((_SSTRIP,), jnp.int32),
            pltpu.VMEM((_SSTRIP,), jnp.int32),
        ],
    )(bins, vals, zeros)


def _vote_body(cnt_ref, out_ref):
    w = [cnt_ref[0, p] for p in range(5)]

    for parity in (0, 1):
        def cnt(c):
            bp = parity * 80 + 4 * c
            return (w[bp >> 5] >> (bp & 31)) & 15

        best = cnt(1)
        bestc = jnp.full_like(best, 1)
        for c in range(2, _NCLASS):
            cc = cnt(c)
            gt = cc > best
            best = jnp.where(gt, cc, best)
            bestc = jnp.where(gt, c, bestc)
        nonempty = (best > 0) | (cnt(0) > 0)
        out_ref[0, parity] = jnp.where(nonempty, bestc, 0)


def _vote_call(counts4):
    return pl.pallas_call(
        _vote_body,
        grid=(_NSC, 4),
        in_specs=[pl.BlockSpec((1, 5, 675, 128), lambda c, j: (c, 0, 0, j))],
        out_specs=pl.BlockSpec((1, 2, 675, 128), lambda c, j: (c, 0, 0, j)),
        out_shape=jax.ShapeDtypeStruct((_NSC, 2, 675, 512), jnp.int32),
    )(counts4)


def kernel(xyz, intensity, labels):
    xyz_t = xyz.T.reshape(3, _ROWS, 128)
    inten2 = intensity.reshape(_ROWS, 128)
    lab2 = labels.reshape(_ROWS, 128)

    fea, grid, bins, vals = _points_call(xyz_t, inten2, lab2)

    zeros = jnp.zeros((_SSTRIP,), jnp.int32)
    counts = _sc_hist_call(bins.reshape(_NSC * _N), vals.reshape(_N),
                           zeros)
    counts4 = counts.reshape(_NSC, 5, 675, 512)
    voted = _vote_call(counts4)
    processed_label = voted.reshape(_GRID)

    return_fea = fea.reshape(9, _N).T
    grid_ind = grid.reshape(3, _N).T
    return return_fea, processed_label, grid_ind
